# baseline (speedup 1.0000x reference)
import functools

import jax
import jax.numpy as jnp
from jax import lax
from jax.experimental import pallas as pl
from jax.experimental.pallas import tpu as pltpu
from jax.experimental.pallas import tpu_sc as plsc

BATCH = 16384
EMBED = 64
NGENRE = 26
OUTD = 2 * EMBED
LANES = 16
GPB = 2


def _tree_sum(terms):
    while len(terms) > 1:
        nxt = [a + b for a, b in zip(terms[::2], terms[1::2])]
        if len(terms) % 2:
            nxt.append(terms[-1])
        terms = nxt
    return terms[0]


@functools.cache
def _make_fused():
    info = plsc.get_sparse_core_info()
    nc, ns = info.num_cores, info.num_subcores
    nw = nc * ns
    bpw = BATCH // nw
    nblk = bpw // (GPB * LANES)
    mesh = plsc.VectorSubcoreMesh(core_axis_name="c", subcore_axis_name="s")

    @functools.partial(
        pl.kernel,
        mesh=mesh,
        out_type=jax.ShapeDtypeStruct((BATCH, OUTD), jnp.float32),
        scratch_types=[
            pltpu.VMEM((bpw,), jnp.int32),
            pltpu.VMEM((bpw, EMBED), jnp.float32),
            pltpu.VMEM((bpw, OUTD), jnp.float32),
            pltpu.VMEM((bpw, NGENRE), jnp.int32),
            pltpu.VMEM((EMBED * NGENRE,), jnp.float32),
            pltpu.SemaphoreType.DMA,
        ],
        compiler_params=pltpu.CompilerParams(
            use_tc_tiling_on_sc=False, needs_layout_passes=False),
    )
    def fused(table_hbm, idx_hbm, g_hbm, w_hbm, out_hbm,
              idx_v, rows_v, comb_v, g_v, w_v, sem):
        wid = lax.axis_index("s") * nc + lax.axis_index("c")
        base = wid * bpw
        pltpu.sync_copy(idx_hbm.at[pl.ds(base, bpw)], idx_v)
        gather = pltpu.async_copy(table_hbm.at[idx_v], rows_v, sem)
        pltpu.sync_copy(g_hbm.at[pl.ds(base, bpw)], g_v)
        pltpu.sync_copy(w_hbm, w_v)

        lane = lax.iota(jnp.int32, 16)
        one = jnp.float32(1.0)

        def block(b, carry):
            r0 = b * (GPB * LANES)
            gcols = []
            invs = []
            rows = []
            for g in range(GPB):
                rg = r0 + g * LANES
                rvec = rg + lane
                cols = [plsc.load_gather(
                            g_v, [rvec, jnp.full((16,), j, jnp.int32)]
                        ).astype(jnp.float32) for j in range(NGENRE)]
                gcols.append(cols)
                invs.append(one / _tree_sum(list(cols)))
                rows.append(rvec)

            def dbody(d, carry2):
                wbase = jnp.broadcast_to(d * NGENRE, (16,)).astype(jnp.int32)
                dcol = jnp.broadcast_to(d + EMBED, (16,)).astype(jnp.int32)
                ws = [plsc.load_gather(w_v, [wbase + j]) for j in range(NGENRE)]
                for g in range(GPB):
                    acc = _tree_sum([gcols[g][j] * ws[j] for j in range(NGENRE)])
                    plsc.store_scatter(comb_v, [rows[g], dcol], acc * invs[g])
                return carry2

            return lax.fori_loop(0, EMBED, dbody, carry, unroll=4)

        lax.fori_loop(0, nblk, block, 0)
        gather.wait()

        def yrow(r, carry):
            for c in range(EMBED // LANES):
                comb_v[r, pl.ds(c * LANES, LANES)] = rows_v[r, pl.ds(c * LANES, LANES)]
            return carry

        lax.fori_loop(0, bpw, yrow, 0, unroll=8)
        pltpu.sync_copy(comb_v, out_hbm.at[pl.ds(base, bpw)])

    return fused


def kernel(year_idx, genre_idx, embedding_year, W_genre):
    idx = year_idx.astype(jnp.int32)
    return _make_fused()(embedding_year, idx, genre_idx, W_genre.reshape(-1))


# speedup vs baseline: 1.0730x; 1.0730x over previous
import functools

import jax
import jax.numpy as jnp
from jax import lax
from jax.experimental import pallas as pl
from jax.experimental.pallas import tpu as pltpu
from jax.experimental.pallas import tpu_sc as plsc

BATCH = 16384
EMBED = 64
NGENRE = 26
OUTD = 2 * EMBED
LANES = 16
GPB = 2
NCHAINS = 4


@functools.cache
def _make_fused():
    info = plsc.get_sparse_core_info()
    nc, ns = info.num_cores, info.num_subcores
    nw = nc * ns
    bpw = BATCH // nw
    nblk = bpw // (GPB * LANES)
    wrow = NGENRE * LANES
    mesh = plsc.VectorSubcoreMesh(core_axis_name="c", subcore_axis_name="s")

    @functools.partial(
        pl.kernel,
        mesh=mesh,
        out_type=jax.ShapeDtypeStruct((BATCH, OUTD), jnp.float32),
        scratch_types=[
            pltpu.VMEM((bpw,), jnp.int32),
            pltpu.VMEM((bpw, EMBED), jnp.float32),
            pltpu.VMEM((bpw, EMBED), jnp.float32),
            pltpu.VMEM((bpw, NGENRE), jnp.int32),
            pltpu.VMEM((EMBED * NGENRE * LANES,), jnp.float32),
            pltpu.SemaphoreType.DMA,
        ],
        compiler_params=pltpu.CompilerParams(
            use_tc_tiling_on_sc=False, needs_layout_passes=False),
    )
    def fused(table_hbm, idx_hbm, g_hbm, wb_hbm, out_hbm,
              idx_v, rows_v, gen_v, g_v, wb_v, sem):
        wid = lax.axis_index("s") * nc + lax.axis_index("c")
        base = wid * bpw
        pltpu.sync_copy(idx_hbm.at[pl.ds(base, bpw)], idx_v)
        gather = pltpu.async_copy(table_hbm.at[idx_v], rows_v, sem)
        pltpu.sync_copy(g_hbm.at[pl.ds(base, bpw)], g_v)
        pltpu.sync_copy(wb_hbm, wb_v)

        lane = lax.iota(jnp.int32, 16)
        one = jnp.float32(1.0)

        def block(b, carry):
            r0 = b * (GPB * LANES)
            gcols = []
            invs = []
            rows = []
            for g in range(GPB):
                rvec = r0 + g * LANES + lane
                cols = [plsc.load_gather(
                            g_v, [rvec, jnp.full((16,), j, jnp.int32)]
                        ).astype(jnp.float32) for j in range(NGENRE)]
                cnt = cols[0] + cols[1]
                for j in range(2, NGENRE):
                    cnt = cnt + cols[j]
                gcols.append(cols)
                invs.append(one / cnt)
                rows.append(rvec)

            def dbody(d, carry2):
                dbase = d * wrow
                dcol = jnp.broadcast_to(d, (16,)).astype(jnp.int32)
                parts = [[None] * NCHAINS for _ in range(GPB)]
                for j in range(NGENRE):
                    w = wb_v[pl.ds(dbase + j * LANES, LANES)]
                    c = j % NCHAINS
                    for g in range(GPB):
                        t = gcols[g][j] * w
                        parts[g][c] = t if parts[g][c] is None else parts[g][c] + t
                for g in range(GPB):
                    p = parts[g]
                    acc = (p[0] + p[1]) + (p[2] + p[3])
                    plsc.store_scatter(gen_v, [rows[g], dcol], acc * invs[g])
                return carry2

            return lax.fori_loop(0, EMBED, dbody, carry)

        lax.fori_loop(0, nblk, block, 0)
        gather.wait()
        pltpu.sync_copy(rows_v, out_hbm.at[pl.ds(base, bpw), pl.ds(0, EMBED)])
        pltpu.sync_copy(gen_v, out_hbm.at[pl.ds(base, bpw), pl.ds(EMBED, EMBED)])

    return fused


def kernel(year_idx, genre_idx, embedding_year, W_genre):
    idx = year_idx.astype(jnp.int32)
    w_big = jnp.broadcast_to(W_genre[:, :, None],
                             (EMBED, NGENRE, LANES)).reshape(-1)
    return _make_fused()(embedding_year, idx, genre_idx, w_big)


# speedup vs baseline: 1.0972x; 1.0226x over previous
import functools

import jax
import jax.numpy as jnp
from jax import lax
from jax.experimental import pallas as pl
from jax.experimental.pallas import tpu as pltpu
from jax.experimental.pallas import tpu_sc as plsc

BATCH = 16384
EMBED = 64
NGENRE = 26
OUTD = 2 * EMBED
LANES = 16
GPB = 2
NCHAINS = 4


@functools.cache
def _make_fused():
    info = plsc.get_sparse_core_info()
    nc, ns = info.num_cores, info.num_subcores
    nw = nc * ns
    bpw = BATCH // nw
    nblk = bpw // (GPB * LANES)
    wrow = NGENRE * LANES
    mesh = plsc.VectorSubcoreMesh(core_axis_name="c", subcore_axis_name="s")

    @functools.partial(
        pl.kernel,
        mesh=mesh,
        out_type=jax.ShapeDtypeStruct((BATCH, OUTD), jnp.float32),
        scratch_types=[
            pltpu.VMEM((bpw,), jnp.int32),
            pltpu.VMEM((bpw, EMBED), jnp.float32),
            pltpu.VMEM((bpw, EMBED), jnp.float32),
            pltpu.VMEM((bpw, NGENRE), jnp.int32),
            pltpu.VMEM((EMBED * NGENRE * LANES,), jnp.float32),
            pltpu.SemaphoreType.DMA,
        ],
        compiler_params=pltpu.CompilerParams(
            use_tc_tiling_on_sc=False, needs_layout_passes=False),
    )
    def fused(table_hbm, idx_hbm, g_hbm, wb_hbm, out_hbm,
              idx_v, rows_v, gen_v, g_v, wb_v, sem):
        wid = lax.axis_index("s") * nc + lax.axis_index("c")
        base = wid * bpw
        pltpu.sync_copy(idx_hbm.at[pl.ds(base, bpw)], idx_v)
        gather = pltpu.async_copy(table_hbm.at[idx_v], rows_v, sem)
        pltpu.sync_copy(g_hbm.at[pl.ds(base, bpw)], g_v)
        pltpu.sync_copy(wb_hbm, wb_v)

        lane = lax.iota(jnp.int32, 16)
        one = jnp.float32(1.0)

        @plsc.parallel_loop(0, nblk)
        def block(b):
            r0 = b * (GPB * LANES)
            gcols = []
            invs = []
            rows = []
            for g in range(GPB):
                rvec = r0 + g * LANES + lane
                cols = [plsc.load_gather(
                            g_v, [rvec, jnp.full((16,), j, jnp.int32)]
                        ).astype(jnp.float32) for j in range(NGENRE)]
                cnt = cols[0] + cols[1]
                for j in range(2, NGENRE):
                    cnt = cnt + cols[j]
                gcols.append(cols)
                invs.append(one / cnt)
                rows.append(rvec)

            @plsc.parallel_loop(0, EMBED, unroll=2)
            def dloop(d):
                dbase = d * wrow
                dcol = jnp.broadcast_to(d, (16,)).astype(jnp.int32)
                parts = [[None] * NCHAINS for _ in range(GPB)]
                for j in range(NGENRE):
                    w = wb_v[pl.ds(dbase + j * LANES, LANES)]
                    c = j % NCHAINS
                    for g in range(GPB):
                        t = gcols[g][j] * w
                        parts[g][c] = t if parts[g][c] is None else parts[g][c] + t
                for g in range(GPB):
                    p = parts[g]
                    acc = (p[0] + p[1]) + (p[2] + p[3])
                    plsc.store_scatter(gen_v, [rows[g], dcol], acc * invs[g])
        gather.wait()
        pltpu.sync_copy(rows_v, out_hbm.at[pl.ds(base, bpw), pl.ds(0, EMBED)])
        pltpu.sync_copy(gen_v, out_hbm.at[pl.ds(base, bpw), pl.ds(EMBED, EMBED)])

    return fused


def kernel(year_idx, genre_idx, embedding_year, W_genre):
    idx = year_idx.astype(jnp.int32)
    w_big = jnp.broadcast_to(W_genre[:, :, None],
                             (EMBED, NGENRE, LANES)).reshape(-1)
    return _make_fused()(embedding_year, idx, genre_idx, w_big)


# speedup vs baseline: 1.6413x; 1.4959x over previous
import functools

import jax
import jax.numpy as jnp
from jax import lax
from jax.experimental import pallas as pl
from jax.experimental.pallas import tpu as pltpu
from jax.experimental.pallas import tpu_sc as plsc

BATCH = 16384
EMBED = 64
NGENRE = 26
OUTD = 2 * EMBED
LANES = 16
GPB = 2
NCHAINS = 4


@functools.cache
def _make_fused():
    info = plsc.get_sparse_core_info()
    nc, ns = info.num_cores, info.num_subcores
    nw = nc * ns
    bpw = BATCH // nw
    nblk = bpw // (GPB * LANES)
    wrow = NGENRE * LANES
    mesh = plsc.VectorSubcoreMesh(core_axis_name="c", subcore_axis_name="s")

    @functools.partial(
        pl.kernel,
        mesh=mesh,
        out_type=jax.ShapeDtypeStruct((BATCH, OUTD), jnp.float32),
        scratch_types=[
            pltpu.VMEM((bpw,), jnp.int32),
            pltpu.VMEM((bpw, EMBED), jnp.float32),
            pltpu.VMEM((bpw, EMBED), jnp.float32),
            pltpu.VMEM((bpw, NGENRE), jnp.int32),
            pltpu.VMEM((EMBED * NGENRE * LANES,), jnp.float32),
            pltpu.SemaphoreType.DMA,
        ],
        compiler_params=pltpu.CompilerParams(
            use_tc_tiling_on_sc=False, needs_layout_passes=False),
    )
    def fused(table_hbm, idx_hbm, g_hbm, wb_hbm, out_hbm,
              idx_v, rows_v, gen_v, g_v, wb_v, sem):
        wid = lax.axis_index("s") * nc + lax.axis_index("c")
        base = wid * bpw
        pltpu.sync_copy(idx_hbm.at[pl.ds(base, bpw)], idx_v)
        gather = pltpu.async_copy(table_hbm.at[idx_v], rows_v, sem)
        pltpu.sync_copy(g_hbm.at[pl.ds(base, bpw)], g_v)
        pltpu.sync_copy(wb_hbm, wb_v)

        lane = lax.iota(jnp.int32, 16)
        one = jnp.float32(1.0)

        def _unused_block(b):
            r0 = b * (GPB * LANES)
            gcols = []
            invs = []
            rows = []
            for g in range(GPB):
                rvec = r0 + g * LANES + lane
                cols = [plsc.load_gather(
                            g_v, [rvec, jnp.full((16,), j, jnp.int32)]
                        ).astype(jnp.float32) for j in range(NGENRE)]
                cnt = cols[0] + cols[1]
                for j in range(2, NGENRE):
                    cnt = cnt + cols[j]
                gcols.append(cols)
                invs.append(one / cnt)
                rows.append(rvec)

            def dloop(d):
                dbase = d * wrow
                dcol = jnp.broadcast_to(d, (16,)).astype(jnp.int32)
                parts = [[None] * NCHAINS for _ in range(GPB)]
                for j in range(NGENRE):
                    w = wb_v[pl.ds(dbase + j * LANES, LANES)]
                    c = j % NCHAINS
                    for g in range(GPB):
                        t = gcols[g][j] * w
                        parts[g][c] = t if parts[g][c] is None else parts[g][c] + t
                for g in range(GPB):
                    p = parts[g]
                    acc = (p[0] + p[1]) + (p[2] + p[3])
                    plsc.store_scatter(gen_v, [rows[g], dcol], acc * invs[g])
        gather.wait()
        pltpu.sync_copy(rows_v, out_hbm.at[pl.ds(base, bpw), pl.ds(0, EMBED)])
        pltpu.sync_copy(gen_v, out_hbm.at[pl.ds(base, bpw), pl.ds(EMBED, EMBED)])

    return fused


def kernel(year_idx, genre_idx, embedding_year, W_genre):
    idx = year_idx.astype(jnp.int32)
    w_big = jnp.broadcast_to(W_genre[:, :, None],
                             (EMBED, NGENRE, LANES)).reshape(-1)
    return _make_fused()(embedding_year, idx, genre_idx, w_big)


# speedup vs baseline: 1.8904x; 1.1517x over previous
import functools

import jax
import jax.numpy as jnp
from jax import lax
from jax.experimental import pallas as pl
from jax.experimental.pallas import tpu as pltpu
from jax.experimental.pallas import tpu_sc as plsc

BATCH = 16384
EMBED = 64
NGENRE = 26
OUTD = 2 * EMBED
LANES = 16


@functools.cache
def _make_sc_gather():
    info = plsc.get_sparse_core_info()
    nc, ns = info.num_cores, info.num_subcores
    nw = nc * ns
    bpw = BATCH // nw
    mesh = plsc.VectorSubcoreMesh(core_axis_name="c", subcore_axis_name="s")

    @functools.partial(
        pl.kernel,
        mesh=mesh,
        out_type=jax.ShapeDtypeStruct((BATCH, EMBED), jnp.float32),
        scratch_types=[
            pltpu.VMEM((bpw,), jnp.int32),
            pltpu.VMEM((bpw, EMBED), jnp.float32),
            pltpu.SemaphoreType.DMA,
        ],
        compiler_params=pltpu.CompilerParams(use_tc_tiling_on_sc=True),
    )
    def sc_gather(table_hbm, idx_hbm, out_hbm, idx_v, rows_v, sem):
        wid = lax.axis_index("s") * nc + lax.axis_index("c")
        base = wid * bpw
        pltpu.sync_copy(idx_hbm.at[pl.ds(base, bpw)], idx_v)

        def chunk(k, carry):
            k16 = k * LANES
            iv = idx_v[pl.ds(k16, LANES)]
            cps = []
            for t in range(LANES):
                cps.append(pltpu.async_copy(
                    table_hbm.at[iv[t]], rows_v.at[k16 + t], sem))
            for cp in cps:
                cp.wait()
            return carry

        lax.fori_loop(0, bpw // LANES, chunk, 0)
        pltpu.sync_copy(rows_v, out_hbm.at[pl.ds(base, bpw)])

    return sc_gather


def _combine_body(year_ref, g_ref, w_ref, out_ref):
    gf = g_ref[...].astype(jnp.float32)
    s = jnp.sum(gf, axis=1, keepdims=True)
    proj = jax.lax.dot_general(
        gf, w_ref[...], (((1,), (1,)), ((), ())),
        preferred_element_type=jnp.float32)
    out_ref[:, :EMBED] = year_ref[...]
    out_ref[:, EMBED:] = proj / s


def _combine(year, g, w):
    grid = 8
    bs = BATCH // grid
    return pl.pallas_call(
        _combine_body,
        grid=(grid,),
        in_specs=[
            pl.BlockSpec((bs, EMBED), lambda i: (i, 0)),
            pl.BlockSpec((bs, NGENRE), lambda i: (i, 0)),
            pl.BlockSpec((EMBED, NGENRE), lambda i: (0, 0)),
        ],
        out_specs=pl.BlockSpec((bs, OUTD), lambda i: (i, 0)),
        out_shape=jax.ShapeDtypeStruct((BATCH, OUTD), jnp.float32),
    )(year, g, w)


def kernel(year_idx, genre_idx, embedding_year, W_genre):
    idx = year_idx.astype(jnp.int32)
    year_emb = _make_sc_gather()(embedding_year, idx)
    return _combine(year_emb, genre_idx, W_genre)
